# Initial kernel scaffold; baseline (speedup 1.0000x reference)
import jax
import jax.numpy as jnp
from jax.experimental import pallas as pl


def kernel(pred, label, seen):
    raise NotImplementedError("write your pallas kernel here")


# speedup vs baseline: 2.4376x; 2.4376x over previous
import functools

import jax
import jax.numpy as jnp
import numpy as np
from jax import lax
from jax.experimental import pallas as pl
from jax.experimental.pallas import tpu as pltpu
from jax.experimental.pallas import tpu_sc as plsc

_ANCHORS = [[42.31, 55.41], [102.17, 128.3], [161.79, 259.17], [303.08, 154.9], [359.56, 320.23]]
_ABW = tuple(float(np.float32(np.float32(a[0]) / np.float32(512.0 / 14.0) / np.float32(14.0))) for a in _ANCHORS)
_ABH = tuple(float(np.float32(np.float32(a[1]) / np.float32(512.0 / 14.0) / np.float32(14.0))) for a in _ANCHORS)
_NEG = float(np.float32(-3e38))
_LN2 = 0.6931471805599453
_BS, _A, _N, _NP = 64, 5, 196, 208
_CHUNKS = _NP // 16


def _ln(x):
    bits = lax.bitcast_convert_type(x, jnp.int32)
    e = ((bits >> 23) & 0xFF) - 127
    m = lax.bitcast_convert_type((bits & 0x007FFFFF) | 0x3F800000, jnp.float32)
    s = (m - 1.0) / (m + 1.0)
    s2 = s * s
    p = 1.0 + s2 * (1.0 / 3.0 + s2 * (0.2 + s2 * (1.0 / 7.0 + s2 * (1.0 / 9.0))))
    ln = e.astype(jnp.float32) * _LN2 + 2.0 * s * p
    return jnp.where(x > 0.0, ln, -jnp.inf)


def _sig(x):
    return 1.0 / (1.0 + jnp.exp(-x))


_GDN = lax.GatherDimensionNumbers(offset_dims=(), collapsed_slice_dims=(0,),
                                  start_index_map=(0,))


def _lane_bcast(v, r):
    ridx = jnp.full((16, 1), r, jnp.int32)
    return lax.gather(v, ridx, _GDN, (1,),
                      mode=lax.GatherScatterMode.PROMISE_IN_BOUNDS)


def _sc_body(imgs_per, nc, pred_h, lab_h, coef_h, out_h,
             pred_v, lab_v, gt_v, sg_v, coef_v, out_v):
    wid = lax.axis_index("s") * nc + lax.axis_index("c")
    pltpu.sync_copy(coef_h, coef_v)
    coefv = coef_v[...]
    lane = lax.iota(jnp.int32, 16)
    zeros16 = jnp.zeros((16,), jnp.float32)

    def one_image(i, total):
        b = wid * imgs_per + i
        pltpu.sync_copy(pred_h.at[b], pred_v)
        pltpu.sync_copy(lab_h.at[b], lab_v)

        def zbody(k, carry):
            sl = pl.ds(k * 16, 16)
            for r in range(30):
                sg_v[r, sl] = zeros16
            return carry

        lax.fori_loop(0, _CHUNKS, zbody, jnp.int32(0))

        def abody(k, carry):
            acc_t, anyv = carry
            sl = pl.ds(k * 16, 16)
            n_i = lane + k * 16
            colf = lax.rem(n_i, 14).astype(jnp.float32)
            rowf = lax.div(n_i, 14).astype(jnp.float32)
            l0 = lab_v[0, sl]
            l3 = lab_v[3, sl]
            l4 = lab_v[4, sl]
            tw = lab_v[5, sl]
            th = lab_v[6, sl]
            valid = l0 != 0.0
            vldf = jnp.where(valid, 1.0, 0.0).astype(jnp.float32)
            txf = l3 + colf
            tyf = l4 + rowf
            wi = txf.astype(jnp.int32)
            hj = tyf.astype(jnp.int32)
            gtx = txf / 14.0
            gty = tyf / 14.0
            gx1 = gtx - 0.5 * tw
            gx2 = gtx + 0.5 * tw
            gy1 = gty - 0.5 * th
            gy2 = gty + 0.5 * th
            ga = tw * th
            gc = jnp.where(valid, -0.6 * ga, _NEG)
            gt_v[0, sl] = gx1
            gt_v[1, sl] = gx2
            gt_v[2, sl] = gy1
            gt_v[3, sl] = gy2
            gt_v[4, sl] = gc
            best = jnp.full((16,), -1.0, jnp.float32)
            bid = jnp.zeros((16,), jnp.int32)
            aws = jnp.full((16,), _ABW[0], jnp.float32)
            ahs = jnp.full((16,), _ABH[0], jnp.float32)
            for a in range(_A):
                inter = jnp.minimum(tw, _ABW[a]) * jnp.minimum(th, _ABH[a])
                u = ga + (_ABW[a] * _ABH[a]) - inter
                iou = jnp.maximum(inter / u, 0.0)
                cnd = iou > best
                best = jnp.where(cnd, iou, best)
                bid = jnp.where(cnd, a, bid)
                aws = jnp.where(cnd, _ABW[a], aws)
                ahs = jnp.where(cnd, _ABH[a], ahs)
            wg = jnp.minimum(wi, 13)
            hg = jnp.minimum(hj, 13)
            cg = hg * 14 + wg
            ch0 = bid * 7
            p0g = plsc.load_gather(pred_v, [ch0, cg])
            p1g = plsc.load_gather(pred_v, [ch0 + 1, cg])
            p2g = plsc.load_gather(pred_v, [ch0 + 2, cg])
            p3g = plsc.load_gather(pred_v, [ch0 + 3, cg])
            p4g = plsc.load_gather(pred_v, [ch0 + 4, cg])
            p5g = plsc.load_gather(pred_v, [ch0 + 5, cg])
            p6g = plsc.load_gather(pred_v, [ch0 + 6, cg])
            row1 = jnp.full((16,), 1, jnp.int32)
            cv1 = plsc.load_gather(lab_v, [row1, cg])
            cv2 = plsc.load_gather(lab_v, [row1 + 1, cg])
            ax = _sig(p3g)
            ay = _sig(p4g)
            lnw = _ln(tw / aws)
            lnh = _ln(th / ahs)
            truth_x = txf - wi.astype(jnp.float32)
            truth_y = tyf - hj.astype(jnp.float32)
            scale = 2.0 - truth_x * truth_y
            px = (ax + wg.astype(jnp.float32)) / 14.0
            py = (ay + hg.astype(jnp.float32)) / 14.0
            pw = jnp.exp(p5g) * aws
            ph = jnp.exp(p6g) * ahs
            px1 = px - 0.5 * pw
            px2 = px + 0.5 * pw
            py1 = py - 0.5 * ph
            py2 = py + 0.5 * ph
            xi1 = jnp.maximum(px1, gx1)
            xi2 = jnp.minimum(px2, gx2)
            yi1 = jnp.maximum(py1, gy1)
            yi2 = jnp.minimum(py2, gy2)
            inter = jnp.maximum(xi2 - xi1, 0.0) * jnp.maximum(yi2 - yi1, 0.0)
            u = pw * ph + ga - inter
            iou_t = jnp.maximum(inter / u, 0.0)
            obj5 = 5.0 * (p0g - iou_t)
            c1 = scale * (ax - truth_x)
            c2 = scale * (ay - truth_y)
            c3 = scale * (p5g - lnw)
            c4 = scale * (p6g - lnh)
            cls = (p1g - cv1) * (p1g - cv1) + (p2g - cv2) * (p2g - cv2)
            inb = valid & (wi <= 13) & (hj <= 13)
            tcell = hg * 14 + wg
            b5 = bid * 5
            onesf = jnp.full((16,), 1.0, jnp.float32)
            plsc.addupdate_scatter(sg_v, [b5, tcell], c1, mask=inb)
            plsc.addupdate_scatter(sg_v, [b5 + 1, tcell], c2, mask=inb)
            plsc.addupdate_scatter(sg_v, [b5 + 2, tcell], c3, mask=inb)
            plsc.addupdate_scatter(sg_v, [b5 + 3, tcell], c4, mask=inb)
            plsc.addupdate_scatter(sg_v, [b5 + 4, tcell], obj5, mask=inb)
            plsc.addupdate_scatter(sg_v, [bid + 25, tcell], onesf, mask=inb)
            acc_t = acc_t + jnp.where(valid, cls, 0.0)
            anyv = jnp.maximum(anyv, vldf)
            return acc_t, anyv

        acc_t, anyv = lax.fori_loop(0, _CHUNKS, abody, (zeros16, zeros16))

        acc_b = zeros16
        for a in range(_A):
            def bbody(k, acc, a=a):
                sl = pl.ds(k * 16, 16)
                n_i = lane + k * 16
                colf = lax.rem(n_i, 14).astype(jnp.float32)
                rowf = lax.div(n_i, 14).astype(jnp.float32)
                p0 = pred_v[7 * a + 0, sl]
                p3 = pred_v[7 * a + 3, sl]
                p4 = pred_v[7 * a + 4, sl]
                p5 = pred_v[7 * a + 5, sl]
                p6 = pred_v[7 * a + 6, sl]
                ax = _sig(p3)
                ay = _sig(p4)
                px = (ax + colf) / 14.0
                py = (ay + rowf) / 14.0
                pw = jnp.exp(p5) * _ABW[a]
                ph = jnp.exp(p6) * _ABH[a]
                px1 = px - 0.5 * pw
                px2 = px + 0.5 * pw
                py1 = py - 0.5 * ph
                py2 = py + 0.5 * ph
                pam = -0.6 * (pw * ph)

                def gbody(cc, m):
                    slg = pl.ds(cc * 16, 16)
                    vx1 = gt_v[0, slg]
                    vx2 = gt_v[1, slg]
                    vy1 = gt_v[2, slg]
                    vy2 = gt_v[3, slg]
                    vgc = gt_v[4, slg]
                    for r in range(16):
                        gx1s = _lane_bcast(vx1, r)
                        gx2s = _lane_bcast(vx2, r)
                        gy1s = _lane_bcast(vy1, r)
                        gy2s = _lane_bcast(vy2, r)
                        gcs = _lane_bcast(vgc, r)
                        t = pam + gcs
                        xi1 = jnp.maximum(px1, gx1s)
                        xi2 = jnp.minimum(px2, gx2s)
                        yi1 = jnp.maximum(py1, gy1s)
                        yi2 = jnp.minimum(py2, gy2s)
                        inter = jnp.maximum(xi2 - xi1, 0.0) * jnp.maximum(yi2 - yi1, 0.0)
                        m = jnp.maximum(m, 1.6 * inter + t)
                    return m

                m = lax.fori_loop(0, _CHUNKS, gbody, jnp.full((16,), _NEG, jnp.float32))
                ol = jnp.where(m <= 0.0, 0.5 * p0, 0.0)
                mc = sg_v[25 + a, sl]
                omc = 1.0 - mc
                for ci, co in enumerate((coefv * (ax - 0.5), coefv * (ay - 0.5),
                                         coefv * p5, coefv * p6, ol)):
                    tg = sg_v[a * 5 + ci, sl]
                    bc = mc * tg + omc * co
                    acc = acc + bc * bc
                return acc

            acc_b = lax.fori_loop(0, _CHUNKS, bbody, acc_b)

        pg = jnp.sum(acc_t) + jnp.sum(acc_b)
        anyok = jnp.max(anyv) > 0.0
        return total + jnp.where(anyok, pg, 0.0)

    total = lax.fori_loop(0, imgs_per, one_image, jnp.float32(0.0))
    out_v[...] = jnp.where(lane == 0, total, 0.0)
    pltpu.sync_copy(out_v, out_h.at[wid])


def kernel(pred, label, seen):
    try:
        info = plsc.get_sparse_core_info()
        nc, ns = info.num_cores, info.num_subcores
    except Exception:
        nc, ns = 2, 16
    nw = nc * ns
    imgs_per = _BS // nw
    pred2 = jnp.pad(pred.reshape(_BS, 35, _N), ((0, 0), (0, 0), (0, _NP - _N)))
    lab2 = jnp.pad(label.reshape(_BS, 7, _N), ((0, 0), (0, 0), (0, _NP - _N)))
    coef = jnp.where(jnp.asarray(seen) < 12800, jnp.float32(0.01),
                     jnp.float32(0.0)) * jnp.ones((16,), jnp.float32)
    mesh = plsc.VectorSubcoreMesh(core_axis_name="c", subcore_axis_name="s",
                                  num_cores=nc, num_subcores=ns)
    body = functools.partial(_sc_body, imgs_per, nc)
    out = pl.kernel(
        body,
        out_type=jax.ShapeDtypeStruct((nw, 16), jnp.float32),
        mesh=mesh,
        compiler_params=pltpu.CompilerParams(use_tc_tiling_on_sc=False,
                                             needs_layout_passes=False),
        scratch_types=[
            pltpu.VMEM((35, _NP), jnp.float32),
            pltpu.VMEM((7, _NP), jnp.float32),
            pltpu.VMEM((5, _NP), jnp.float32),
            pltpu.VMEM((30, _NP), jnp.float32),
            pltpu.VMEM((16,), jnp.float32),
            pltpu.VMEM((16,), jnp.float32),
        ],
    )(pred2, lab2, coef)
    return (jnp.sum(out) / _BS).reshape(1)
